# baseline (speedup 1.0000x reference)
import jax
import jax.numpy as jnp
from jax.experimental import pallas as pl
from jax.experimental.pallas import tpu as pltpu

_CHUNK = 512


def _embed_kernel(ids_ref, tt_ref, seg_ref, pe_ref, tok_hbm_ref, out_ref,
                  tok_buf, sems):
    core = pl.program_id(0)
    b2 = pl.program_id(1)
    nb2 = pl.num_programs(1)
    S, D = pe_ref.shape
    n_chunks = S // _CHUNK

    row = core * nb2 + b2
    cur = jax.lax.rem(b2, 2)

    def issue_chunk(r_batch, c, buf_set):
        base = c * _CHUNK
        slot = buf_set * n_chunks + c
        for r in range(_CHUNK):
            pltpu.make_async_copy(
                tok_hbm_ref.at[pl.ds(ids_ref[r_batch, base + r], 1), :],
                tok_buf.at[slot, pl.ds(r, 1), :],
                sems.at[slot]).start(priority=0)

    def wait_chunk(slot):
        pltpu.make_async_copy(
            tok_hbm_ref.at[pl.ds(0, _CHUNK), :],
            tok_buf.at[slot, pl.ds(0, _CHUNK), :],
            sems.at[slot]).wait()

    @pl.when(b2 == 0)
    def _():
        for c in range(n_chunks):
            issue_chunk(row, c, cur)

    for c in range(n_chunks):
        @pl.when(b2 + 1 < nb2)
        def _(c=c):
            issue_chunk(row + 1, c, 1 - cur)

        slot = cur * n_chunks + c
        wait_chunk(slot)
        off = c * _CHUNK
        tok = tok_buf[slot]
        tt = tt_ref[0, pl.ds(off, _CHUNK), :]
        seg = jnp.where(tt == 0, seg_ref[0:1, :], seg_ref[1:2, :])
        out_ref[0, pl.ds(off, _CHUNK), :] = tok + pe_ref[pl.ds(off, _CHUNK), :] + seg


def kernel(input_ids, token_type_ids, tok_table, seg_table, pe):
    B, S = input_ids.shape
    V, D = tok_table.shape
    T = seg_table.shape[0]
    n_chunks = S // _CHUNK
    nb2 = B // 2

    if token_type_ids is None:
        token_type_ids = jnp.zeros_like(input_ids)
    ids = input_ids.astype(jnp.int32)
    tt_3d = token_type_ids.astype(jnp.int32).reshape(B, S, 1)

    grid_spec = pltpu.PrefetchScalarGridSpec(
        num_scalar_prefetch=1,
        grid=(2, nb2),
        in_specs=[
            pl.BlockSpec((1, S, 1), lambda i, b2, ids_ref: (i * nb2 + b2, 0, 0)),
            pl.BlockSpec((T, D), lambda i, b2, ids_ref: (0, 0)),
            pl.BlockSpec((S, D), lambda i, b2, ids_ref: (0, 0)),
            pl.BlockSpec(memory_space=pl.ANY),
        ],
        out_specs=pl.BlockSpec((1, S, D), lambda i, b2, ids_ref: (i * nb2 + b2, 0, 0)),
        scratch_shapes=[
            pltpu.VMEM((2 * n_chunks, _CHUNK, D), jnp.float32),
            pltpu.SemaphoreType.DMA((2 * n_chunks,)),
        ],
    )
    return pl.pallas_call(
        _embed_kernel,
        out_shape=jax.ShapeDtypeStruct((B, S, D), jnp.float32),
        grid_spec=grid_spec,
        compiler_params=pltpu.CompilerParams(
            dimension_semantics=("parallel", "arbitrary"),
            disable_bounds_checks=True,
        ),
    )(ids, tt_3d, seg_table, pe, tok_table)


# speedup vs baseline: 1.0047x; 1.0047x over previous
import jax
import jax.numpy as jnp
from jax.experimental import pallas as pl
from jax.experimental.pallas import tpu as pltpu

_ROWS = 2


def _embed_kernel(ids_ref, tt_ref, seg_ref, pe_ref, tok_hbm_ref, out_ref,
                  tok_buf, sems):
    core = pl.program_id(0)
    b2 = pl.program_id(1)
    nb2 = pl.num_programs(1)
    S, D = pe_ref.shape

    base_row = (core * nb2 + b2) * _ROWS
    cur = jax.lax.rem(b2, 2)

    def issue_row(step_base_row, c, buf_set):
        slot = buf_set * _ROWS + c
        for r in range(S):
            pltpu.make_async_copy(
                tok_hbm_ref.at[pl.ds(ids_ref[step_base_row + c, r], 1), :],
                tok_buf.at[slot, pl.ds(r, 1), :],
                sems.at[slot]).start()

    def wait_row(slot):
        pltpu.make_async_copy(
            tok_hbm_ref.at[pl.ds(0, S), :],
            tok_buf.at[slot, pl.ds(0, S), :],
            sems.at[slot]).wait()

    @pl.when(b2 == 0)
    def _():
        for c in range(_ROWS):
            issue_row(base_row, c, cur)

    for c in range(_ROWS):
        @pl.when(b2 + 1 < nb2)
        def _(c=c):
            issue_row(base_row + _ROWS, c, 1 - cur)

        slot = cur * _ROWS + c
        wait_row(slot)
        tok = tok_buf[slot]
        tt = tt_ref[c, :, :]
        seg = jnp.where(tt == 0, seg_ref[0:1, :], seg_ref[1:2, :])
        out_ref[c, :, :] = tok + pe_ref[...] + seg


def kernel(input_ids, token_type_ids, tok_table, seg_table, pe):
    B, S = input_ids.shape
    V, D = tok_table.shape
    T = seg_table.shape[0]
    nb2 = B // 2 // _ROWS

    if token_type_ids is None:
        token_type_ids = jnp.zeros_like(input_ids)
    ids = input_ids.astype(jnp.int32)
    tt_3d = token_type_ids.astype(jnp.int32).reshape(B, S, 1)

    grid_spec = pltpu.PrefetchScalarGridSpec(
        num_scalar_prefetch=1,
        grid=(2, nb2),
        in_specs=[
            pl.BlockSpec((_ROWS, S, 1), lambda i, b2, ids_ref: (i * nb2 + b2, 0, 0)),
            pl.BlockSpec((T, D), lambda i, b2, ids_ref: (0, 0)),
            pl.BlockSpec((S, D), lambda i, b2, ids_ref: (0, 0)),
            pl.BlockSpec(memory_space=pl.ANY),
        ],
        out_specs=pl.BlockSpec((_ROWS, S, D),
                               lambda i, b2, ids_ref: (i * nb2 + b2, 0, 0)),
        scratch_shapes=[
            pltpu.VMEM((2 * _ROWS, S, D), jnp.float32),
            pltpu.SemaphoreType.DMA((2 * _ROWS,)),
        ],
    )
    return pl.pallas_call(
        _embed_kernel,
        out_shape=jax.ShapeDtypeStruct((B, S, D), jnp.float32),
        grid_spec=grid_spec,
        compiler_params=pltpu.CompilerParams(
            dimension_semantics=("parallel", "arbitrary"),
            disable_bounds_checks=True,
        ),
    )(ids, tt_3d, seg_table, pe, tok_table)
